# baseline (speedup 1.0000x reference)
import functools

import jax
import jax.numpy as jnp
from jax import lax
from jax.experimental import pallas as pl
from jax.experimental.pallas import tpu as pltpu
from jax.experimental.pallas import tpu_sc as plsc

N = 10000
E = 320000
D = 128
K = 2

NC = 2
NS = 16
NW = NC * NS
L = 16

CPW = D // NW
CHUNK = 10000
NCHUNK = E // CHUNK

DEG_ROWS = 640
DEG_PAD = 16 * DEG_ROWS

def _worker_id():
  return lax.axis_index("s") * NC + lax.axis_index("c")


@functools.lru_cache(maxsize=None)
def _make_segsum_t():
  mesh = plsc.VectorSubcoreMesh(core_axis_name="c", subcore_axis_name="s")
  return functools.partial(
      pl.kernel,
      out_type=jax.ShapeDtypeStruct((D, N), jnp.float32),
      mesh=mesh,
      scratch_types=[
          pltpu.VMEM((CPW, N), jnp.float32),
          pltpu.VMEM((CPW, N), jnp.float32),
          pltpu.VMEM((2, CHUNK), jnp.int32),
          pltpu.VMEM((2, CHUNK), jnp.int32),
          pltpu.SemaphoreType.DMA,
          pltpu.SemaphoreType.DMA,
      ],
      compiler_params=pltpu.CompilerParams(needs_layout_passes=False,
                                           use_tc_tiling_on_sc=False),
  )(_segsum_t_body)


def _segsum_t_body(xt_hbm, src_hbm, dst_hbm, out_hbm, xs, acc, sbuf, dbuf,
                   ssem, dsem):
  wid = _worker_id()
  c0 = wid * CPW

  pltpu.sync_copy(xt_hbm.at[pl.ds(c0, CPW), :], xs)

  zero = jnp.zeros((L,), jnp.float32)

  def zbody(i, _):
    for c in range(CPW):
      acc[c, pl.ds(i * L, L)] = zero
    return 0

  lax.fori_loop(0, N // L, zbody, 0)

  def start_fetch(buf, j):
    pltpu.make_async_copy(src_hbm.at[pl.ds(j * CHUNK, CHUNK)], sbuf.at[buf],
                          ssem).start()
    pltpu.make_async_copy(dst_hbm.at[pl.ds(j * CHUNK, CHUNK)], dbuf.at[buf],
                          dsem).start()

  def wait_fetch(buf, j):
    pltpu.make_async_copy(src_hbm.at[pl.ds(j * CHUNK, CHUNK)], sbuf.at[buf],
                          ssem).wait()
    pltpu.make_async_copy(dst_hbm.at[pl.ds(j * CHUNK, CHUNK)], dbuf.at[buf],
                          dsem).wait()

  cvecs = [jnp.full((L,), c, jnp.int32) for c in range(CPW)]

  def inner(buf):
    @plsc.parallel_loop(0, CHUNK // L, unroll=16)
    def _(i):
      off = i * L
      s16 = sbuf[buf, pl.ds(off, L)]
      d16 = dbuf[buf, pl.ds(off, L)]
      for c in range(CPW):
        v = plsc.load_gather(xs, [cvecs[c], s16])
        plsc.addupdate_scatter(acc, [cvecs[c], d16], v)

  start_fetch(0, 0)

  def pair_body(jp, _):
    for b in range(2):
      j = jp * 2 + b

      @pl.when(j + 1 < NCHUNK)
      def _():
        start_fetch(1 - b, j + 1)

      wait_fetch(b, j)
      inner(b)
    return 0

  lax.fori_loop(0, NCHUNK // 2, pair_body, 0)

  pltpu.sync_copy(acc, out_hbm.at[pl.ds(c0, CPW), :])


@functools.lru_cache(maxsize=None)
def _make_degrees():
  mesh = plsc.VectorSubcoreMesh(core_axis_name="c", subcore_axis_name="s")
  return functools.partial(
      pl.kernel,
      out_type=[
          jax.ShapeDtypeStruct((DEG_PAD,), jnp.float32),
          jax.ShapeDtypeStruct((DEG_PAD,), jnp.float32),
      ],
      mesh=mesh,
      scratch_types=[
          pltpu.VMEM((DEG_ROWS,), jnp.float32),
          pltpu.VMEM((2, CHUNK), jnp.int32),
          pltpu.SemaphoreType.DMA,
      ],
      compiler_params=pltpu.CompilerParams(needs_layout_passes=False,
                                           use_tc_tiling_on_sc=False),
  )(_degrees_body)


def _degrees_body(di_hbm, du_hbm, degi_hbm, degu_hbm, acc, buf, sem):
  wid = _worker_id()
  grp = wid // 16
  slot = wid % 16
  lo = slot * DEG_ROWS

  zero = jnp.zeros((L,), jnp.float32)
  ones = jnp.full((L,), 1.0, jnp.float32)
  lo_v = jnp.full((L,), 1, jnp.int32) * lo
  hi_v = lo_v + DEG_ROWS

  def scan(idx_hbm, out_hbm):
    def zbody(i, _):
      acc[pl.ds(i * L, L)] = zero
      return 0

    lax.fori_loop(0, DEG_ROWS // L, zbody, 0)

    def start_fetch(b, j):
      pltpu.make_async_copy(idx_hbm.at[pl.ds(j * CHUNK, CHUNK)], buf.at[b],
                            sem).start()

    def wait_fetch(b, j):
      pltpu.make_async_copy(idx_hbm.at[pl.ds(j * CHUNK, CHUNK)], buf.at[b],
                            sem).wait()

    def inner(b):
      @plsc.parallel_loop(0, CHUNK // L, unroll=8)
      def _(i):
        d16 = buf[b, pl.ds(i * L, L)]
        m = (d16 >= lo_v) & (d16 < hi_v)
        idx = jnp.where(m, d16 - lo_v, 0)
        plsc.addupdate_scatter(acc, [idx], ones, mask=m)

    start_fetch(0, 0)

    def pair_body(jp, _):
      for b in range(2):
        j = jp * 2 + b

        @pl.when(j + 1 < NCHUNK)
        def _():
          start_fetch(1 - b, j + 1)

        wait_fetch(b, j)
        inner(b)
      return 0

    lax.fori_loop(0, NCHUNK // 2, pair_body, 0)
    pltpu.sync_copy(acc, out_hbm.at[pl.ds(lo, DEG_ROWS)])

  @pl.when(grp == 0)
  def _():
    scan(di_hbm, degi_hbm)

  @pl.when(grp == 1)
  def _():
    scan(du_hbm, degu_hbm)


BLK = 1000


def _tc_layer_body(hu, hi, mi, mu, degi, degu, Wcf, bcf, Wsf, bsf, Wtf0, btf0,
                   Wtf1, btf1, Wcb, bcb, Wsb, bsb, Wtb0, btb0, Wtb1, btb1,
                   Wcu0, Wcu1, bcu, Wci0, Wci1, bci, out_u, out_i):
  prec = lax.Precision.HIGHEST

  def mm(a, w):
    return jnp.dot(a, w[...], preferred_element_type=jnp.float32,
                   precision=prec)

  hu_b = hu[...]
  hi_b = hi[...]
  inv_i = 1.0 / jnp.maximum(degi[...], 1.0)
  inv_u = 1.0 / jnp.maximum(degu[...], 1.0)

  conv_i = mm(mi[...] * inv_i, Wcf) + bcf[...]
  fi = mm(jax.nn.relu(mm(hi_b, Wsf) + bsf[...] + conv_i), Wtf1) + btf1[...]
  fu = mm(jax.nn.relu(hu_b), Wtf0) + btf0[...]

  conv_u = mm(mu[...] * inv_u, Wcb) + bcb[...]
  bu = mm(jax.nn.relu(mm(hu_b, Wsb) + bsb[...] + conv_u), Wtb0) + btb0[...]
  bi = mm(jax.nn.relu(hi_b), Wtb1) + btb1[...]

  out_u[...] = mm(fu, Wcu0) + mm(bu, Wcu1) + bcu[...]
  out_i[...] = mm(fi, Wci0) + mm(bi, Wci1) + bci[...]


def _tc_layer(hu, hi, mi, mu, degi, degu, weights):
  nblk = pl.BlockSpec((BLK, D), lambda j: (j, 0))
  dspec = pl.BlockSpec((BLK, 1), lambda j: (j, 0))
  wspec = pl.BlockSpec((D, D), lambda j: (0, 0))
  bspec = pl.BlockSpec((1, D), lambda j: (0, 0))
  in_specs = [nblk, nblk, nblk, nblk, dspec, dspec] + [
      wspec if w.shape == (D, D) else bspec for w in weights
  ]
  return pl.pallas_call(
      _tc_layer_body,
      grid=(N // BLK,),
      in_specs=in_specs,
      out_specs=[nblk, nblk],
      out_shape=[
          jax.ShapeDtypeStruct((N, D), jnp.float32),
          jax.ShapeDtypeStruct((N, D), jnp.float32),
      ],
  )(hu, hi, mi, mu, degi, degu, *weights)


def kernel(x_user, x_item, ei_u2i, ei_i2u, Wc_f, bc_f, Ws_f, bs_f, Wc_b, bc_b,
           Ws_b, bs_b, Wt_f, bt_f, Wt_b, bt_b, Wcat, bcat):
  si, di = ei_u2i[0], ei_u2i[1]
  su, du = ei_i2u[0], ei_i2u[1]

  degi_p, degu_p = _make_degrees()(di, du)
  degi = degi_p[:N].reshape(N, 1)
  degu = degu_p[:N].reshape(N, 1)

  hu, hi = x_user, x_item
  for k in range(K):
    segsum_t = _make_segsum_t()
    mi = segsum_t(hu.T, si, di).T
    mu = segsum_t(hi.T, su, du).T
    weights = [
        Wc_f[k], bc_f[k].reshape(1, D),
        Ws_f[k], bs_f[k].reshape(1, D),
        Wt_f[k, 0], bt_f[k, 0].reshape(1, D),
        Wt_f[k, 1], bt_f[k, 1].reshape(1, D),
        Wc_b[k], bc_b[k].reshape(1, D),
        Ws_b[k], bs_b[k].reshape(1, D),
        Wt_b[k, 0], bt_b[k, 0].reshape(1, D),
        Wt_b[k, 1], bt_b[k, 1].reshape(1, D),
        Wcat[k, 0][:D], Wcat[k, 0][D:], bcat[k, 0].reshape(1, D),
        Wcat[k, 1][:D], Wcat[k, 1][D:], bcat[k, 1].reshape(1, D),
    ]
    hu, hi = _tc_layer(hu, hi, mi, mu, degi, degu, weights)
  return jnp.stack([hu, hi])


# speedup vs baseline: 1.2330x; 1.2330x over previous
import functools

import jax
import jax.numpy as jnp
from jax import lax
from jax.experimental import pallas as pl
from jax.experimental.pallas import tpu as pltpu
from jax.experimental.pallas import tpu_sc as plsc

N = 10000
E = 320000
D = 128
K = 2

NC = 2
NS = 16
NW = NC * NS
L = 16

CPW = D // NW
CHUNK = 10000
NCHUNK = E // CHUNK

DEG_ROWS = 640
DEG_PAD = 16 * DEG_ROWS

def _worker_id():
  return lax.axis_index("s") * NC + lax.axis_index("c")


@functools.lru_cache(maxsize=None)
def _make_segsum_t():
  mesh = plsc.VectorSubcoreMesh(core_axis_name="c", subcore_axis_name="s")
  return functools.partial(
      pl.kernel,
      out_type=jax.ShapeDtypeStruct((D, N), jnp.float32),
      mesh=mesh,
      scratch_types=[
          pltpu.VMEM((CPW, N), jnp.float32),
          pltpu.VMEM((CPW, N), jnp.float32),
          pltpu.VMEM((2, CHUNK), jnp.int32),
          pltpu.VMEM((2, CHUNK), jnp.int32),
          pltpu.SemaphoreType.DMA,
          pltpu.SemaphoreType.DMA,
      ],
      compiler_params=pltpu.CompilerParams(needs_layout_passes=False,
                                           use_tc_tiling_on_sc=False),
  )(_segsum_t_body)


def _segsum_t_body(xt_hbm, src_hbm, dst_hbm, out_hbm, xs, acc, sbuf, dbuf,
                   ssem, dsem):
  wid = _worker_id()
  c0 = wid * CPW

  pltpu.sync_copy(xt_hbm.at[pl.ds(c0, CPW), :], xs)

  zero = jnp.zeros((L,), jnp.float32)

  def zbody(i, _):
    for c in range(CPW):
      acc[c, pl.ds(i * L, L)] = zero
    return 0

  lax.fori_loop(0, N // L, zbody, 0)

  def start_fetch(buf, j):
    pltpu.make_async_copy(src_hbm.at[pl.ds(j * CHUNK, CHUNK)], sbuf.at[buf],
                          ssem).start()
    pltpu.make_async_copy(dst_hbm.at[pl.ds(j * CHUNK, CHUNK)], dbuf.at[buf],
                          dsem).start()

  def wait_fetch(buf, j):
    pltpu.make_async_copy(src_hbm.at[pl.ds(j * CHUNK, CHUNK)], sbuf.at[buf],
                          ssem).wait()
    pltpu.make_async_copy(dst_hbm.at[pl.ds(j * CHUNK, CHUNK)], dbuf.at[buf],
                          dsem).wait()

  cvecs = [jnp.full((L,), c, jnp.int32) for c in range(CPW)]

  def inner(buf):
    @plsc.parallel_loop(0, CHUNK // L, unroll=8)
    def _(i):
      off = i * L
      s16 = sbuf[buf, pl.ds(off, L)]
      d16 = dbuf[buf, pl.ds(off, L)]
      for c in range(CPW):
        v = plsc.load_gather(xs, [cvecs[c], s16])
        plsc.addupdate_scatter(acc, [cvecs[c], d16], v)

  start_fetch(0, 0)

  def pair_body(jp, _):
    for b in range(2):
      j = jp * 2 + b

      @pl.when(j + 1 < NCHUNK)
      def _():
        start_fetch(1 - b, j + 1)

      wait_fetch(b, j)
      inner(b)
    return 0

  lax.fori_loop(0, NCHUNK // 2, pair_body, 0)

  pltpu.sync_copy(acc, out_hbm.at[pl.ds(c0, CPW), :])


@functools.lru_cache(maxsize=None)
def _make_degrees():
  mesh = plsc.VectorSubcoreMesh(core_axis_name="c", subcore_axis_name="s")
  return functools.partial(
      pl.kernel,
      out_type=[
          jax.ShapeDtypeStruct((DEG_PAD,), jnp.float32),
          jax.ShapeDtypeStruct((DEG_PAD,), jnp.float32),
      ],
      mesh=mesh,
      scratch_types=[
          pltpu.VMEM((DEG_ROWS,), jnp.float32),
          pltpu.VMEM((2, CHUNK), jnp.int32),
          pltpu.SemaphoreType.DMA,
      ],
      compiler_params=pltpu.CompilerParams(needs_layout_passes=False,
                                           use_tc_tiling_on_sc=False),
  )(_degrees_body)


def _degrees_body(di_hbm, du_hbm, degi_hbm, degu_hbm, acc, buf, sem):
  wid = _worker_id()
  grp = wid // 16
  slot = wid % 16
  lo = slot * DEG_ROWS

  zero = jnp.zeros((L,), jnp.float32)
  ones = jnp.full((L,), 1.0, jnp.float32)
  lo_v = jnp.full((L,), 1, jnp.int32) * lo
  hi_v = lo_v + DEG_ROWS

  def scan(idx_hbm, out_hbm):
    def zbody(i, _):
      acc[pl.ds(i * L, L)] = zero
      return 0

    lax.fori_loop(0, DEG_ROWS // L, zbody, 0)

    def start_fetch(b, j):
      pltpu.make_async_copy(idx_hbm.at[pl.ds(j * CHUNK, CHUNK)], buf.at[b],
                            sem).start()

    def wait_fetch(b, j):
      pltpu.make_async_copy(idx_hbm.at[pl.ds(j * CHUNK, CHUNK)], buf.at[b],
                            sem).wait()

    def inner(b):
      @plsc.parallel_loop(0, CHUNK // L, unroll=8)
      def _(i):
        d16 = buf[b, pl.ds(i * L, L)]
        m = (d16 >= lo_v) & (d16 < hi_v)
        idx = jnp.where(m, d16 - lo_v, 0)
        plsc.addupdate_scatter(acc, [idx], ones, mask=m)

    start_fetch(0, 0)

    def pair_body(jp, _):
      for b in range(2):
        j = jp * 2 + b

        @pl.when(j + 1 < NCHUNK)
        def _():
          start_fetch(1 - b, j + 1)

        wait_fetch(b, j)
        inner(b)
      return 0

    lax.fori_loop(0, NCHUNK // 2, pair_body, 0)
    pltpu.sync_copy(acc, out_hbm.at[pl.ds(lo, DEG_ROWS)])

  @pl.when(grp == 0)
  def _():
    scan(di_hbm, degi_hbm)

  @pl.when(grp == 1)
  def _():
    scan(du_hbm, degu_hbm)


BLK = 1000


def _tc_layer_body(hu, hi, mi, mu, degi, degu, Wfs, bfs, Wbs, bbs, Wtu, btu,
                   Wti, bti, Wcu, bcu, Wci, bci, out_u, out_i):
  prec = lax.Precision.HIGHEST

  def mm(a, w):
    return jnp.dot(a, w[...], preferred_element_type=jnp.float32,
                   precision=prec)

  hu_b = hu[...]
  hi_b = hi[...]
  inv_i = 1.0 / jnp.maximum(degi[...], 1.0)
  inv_u = 1.0 / jnp.maximum(degu[...], 1.0)

  fi0 = mm(jnp.concatenate([mi[...] * inv_i, hi_b], axis=1), Wfs) + bfs[...]
  bu0 = mm(jnp.concatenate([mu[...] * inv_u, hu_b], axis=1), Wbs) + bbs[...]
  tu = mm(jnp.concatenate([jax.nn.relu(hu_b), jax.nn.relu(bu0)], axis=1),
          Wtu) + btu[...]
  ti = mm(jnp.concatenate([jax.nn.relu(fi0), jax.nn.relu(hi_b)], axis=1),
          Wti) + bti[...]
  out_u[...] = mm(tu, Wcu) + bcu[...]
  out_i[...] = mm(ti, Wci) + bci[...]


def _tc_layer(hu, hi, mi, mu, degi, degu, weights):
  nblk = pl.BlockSpec((BLK, D), lambda j: (j, 0))
  dspec = pl.BlockSpec((BLK, 1), lambda j: (j, 0))
  in_specs = [nblk, nblk, nblk, nblk, dspec, dspec] + [
      pl.BlockSpec(w.shape, lambda j: (0, 0)) for w in weights
  ]
  return pl.pallas_call(
      _tc_layer_body,
      grid=(N // BLK,),
      in_specs=in_specs,
      out_specs=[nblk, nblk],
      out_shape=[
          jax.ShapeDtypeStruct((N, D), jnp.float32),
          jax.ShapeDtypeStruct((N, D), jnp.float32),
      ],
  )(hu, hi, mi, mu, degi, degu, *weights)


def kernel(x_user, x_item, ei_u2i, ei_i2u, Wc_f, bc_f, Ws_f, bs_f, Wc_b, bc_b,
           Ws_b, bs_b, Wt_f, bt_f, Wt_b, bt_b, Wcat, bcat):
  si, di = ei_u2i[0], ei_u2i[1]
  su, du = ei_i2u[0], ei_i2u[1]

  degi_p, degu_p = _make_degrees()(di, du)
  degi = degi_p[:N].reshape(N, 1)
  degu = degu_p[:N].reshape(N, 1)

  hu, hi = x_user, x_item
  for k in range(K):
    segsum_t = _make_segsum_t()
    mi = segsum_t(hu.T, si, di).T
    mu = segsum_t(hi.T, su, du).T
    zz = jnp.zeros((D, D), jnp.float32)
    weights = [
        jnp.concatenate([Wc_f[k], Ws_f[k]], axis=0),
        (bc_f[k] + bs_f[k]).reshape(1, D),
        jnp.concatenate([Wc_b[k], Ws_b[k]], axis=0),
        (bc_b[k] + bs_b[k]).reshape(1, D),
        jnp.block([[Wt_f[k, 0], zz], [zz, Wt_b[k, 0]]]),
        jnp.concatenate([bt_f[k, 0], bt_b[k, 0]]).reshape(1, 2 * D),
        jnp.block([[Wt_f[k, 1], zz], [zz, Wt_b[k, 1]]]),
        jnp.concatenate([bt_f[k, 1], bt_b[k, 1]]).reshape(1, 2 * D),
        Wcat[k, 0], bcat[k, 0].reshape(1, D),
        Wcat[k, 1], bcat[k, 1].reshape(1, D),
    ]
    hu, hi = _tc_layer(hu, hi, mi, mu, degi, degu, weights)
  return jnp.stack([hu, hi])


# speedup vs baseline: 1.4397x; 1.1676x over previous
import functools

import jax
import jax.numpy as jnp
from jax import lax
from jax.experimental import pallas as pl
from jax.experimental.pallas import tpu as pltpu
from jax.experimental.pallas import tpu_sc as plsc

N = 10000
E = 320000
D = 128
K = 2

NC = 2
NS = 16
NW = NC * NS
L = 16

CPW = D // NW
CHUNK = 10000
NCHUNK = E // CHUNK

DEG_ROWS = 640
DEG_PAD = 16 * DEG_ROWS

def _worker_id():
  return lax.axis_index("s") * NC + lax.axis_index("c")


PPW = CPW // 2


@functools.lru_cache(maxsize=None)
def _make_segsum_t():
  mesh = plsc.VectorSubcoreMesh(core_axis_name="c", subcore_axis_name="s")
  return functools.partial(
      pl.kernel,
      out_type=jax.ShapeDtypeStruct((D, N), jnp.float32),
      mesh=mesh,
      scratch_types=[
          pltpu.VMEM((PPW, N), jnp.int32),
          pltpu.VMEM((CPW, N), jnp.float32),
          pltpu.VMEM((2, CHUNK), jnp.int32),
          pltpu.SemaphoreType.DMA,
      ],
      compiler_params=pltpu.CompilerParams(needs_layout_passes=False,
                                           use_tc_tiling_on_sc=False),
  )(_segsum_t_body)


def _segsum_t_body(xp_hbm, edge_hbm, out_hbm, xs, acc, ebuf, esem):
  wid = _worker_id()
  c0 = wid * CPW

  pltpu.sync_copy(xp_hbm.at[pl.ds(wid * PPW, PPW), :], xs)

  zero = jnp.zeros((L,), jnp.float32)

  def zbody(i, _):
    for c in range(CPW):
      acc[c, pl.ds(i * L, L)] = zero
    return 0

  lax.fori_loop(0, N // L, zbody, 0)

  def start_fetch(buf, j):
    pltpu.make_async_copy(edge_hbm.at[pl.ds(j * CHUNK, CHUNK)], ebuf.at[buf],
                          esem).start()

  def wait_fetch(buf, j):
    pltpu.make_async_copy(edge_hbm.at[pl.ds(j * CHUNK, CHUNK)], ebuf.at[buf],
                          esem).wait()

  pvecs = [jnp.full((L,), p, jnp.int32) for p in range(PPW)]
  cvecs = [jnp.full((L,), c, jnp.int32) for c in range(CPW)]
  lo16 = jnp.full((L,), 0xFFFF, jnp.int32)
  hi16 = jnp.full((L,), -65536, jnp.int32)

  def inner(buf):
    @plsc.parallel_loop(0, CHUNK // L, unroll=8)
    def _(i):
      w = ebuf[buf, pl.ds(i * L, L)]
      s16 = w & lo16
      d16 = lax.shift_right_logical(w, 16)
      for p in range(PPW):
        g = plsc.load_gather(xs, [pvecs[p], s16])
        vlo = plsc.bitcast(lax.shift_left(g, 16), jnp.float32)
        vhi = plsc.bitcast(g & hi16, jnp.float32)
        plsc.addupdate_scatter(acc, [cvecs[2 * p], d16], vlo)
        plsc.addupdate_scatter(acc, [cvecs[2 * p + 1], d16], vhi)

  start_fetch(0, 0)

  def pair_body(jp, _):
    for b in range(2):
      j = jp * 2 + b

      @pl.when(j + 1 < NCHUNK)
      def _():
        start_fetch(1 - b, j + 1)

      wait_fetch(b, j)
      inner(b)
    return 0

  lax.fori_loop(0, NCHUNK // 2, pair_body, 0)

  pltpu.sync_copy(acc, out_hbm.at[pl.ds(c0, CPW), :])


@functools.lru_cache(maxsize=None)
def _make_degrees():
  mesh = plsc.VectorSubcoreMesh(core_axis_name="c", subcore_axis_name="s")
  return functools.partial(
      pl.kernel,
      out_type=[
          jax.ShapeDtypeStruct((DEG_PAD,), jnp.float32),
          jax.ShapeDtypeStruct((DEG_PAD,), jnp.float32),
      ],
      mesh=mesh,
      scratch_types=[
          pltpu.VMEM((DEG_ROWS,), jnp.float32),
          pltpu.VMEM((2, CHUNK), jnp.int32),
          pltpu.SemaphoreType.DMA,
      ],
      compiler_params=pltpu.CompilerParams(needs_layout_passes=False,
                                           use_tc_tiling_on_sc=False),
  )(_degrees_body)


def _degrees_body(di_hbm, du_hbm, degi_hbm, degu_hbm, acc, buf, sem):
  wid = _worker_id()
  grp = wid // 16
  slot = wid % 16
  lo = slot * DEG_ROWS

  zero = jnp.zeros((L,), jnp.float32)
  ones = jnp.full((L,), 1.0, jnp.float32)
  lo_v = jnp.full((L,), 1, jnp.int32) * lo
  hi_v = lo_v + DEG_ROWS

  def scan(idx_hbm, out_hbm):
    def zbody(i, _):
      acc[pl.ds(i * L, L)] = zero
      return 0

    lax.fori_loop(0, DEG_ROWS // L, zbody, 0)

    def start_fetch(b, j):
      pltpu.make_async_copy(idx_hbm.at[pl.ds(j * CHUNK, CHUNK)], buf.at[b],
                            sem).start()

    def wait_fetch(b, j):
      pltpu.make_async_copy(idx_hbm.at[pl.ds(j * CHUNK, CHUNK)], buf.at[b],
                            sem).wait()

    def inner(b):
      @plsc.parallel_loop(0, CHUNK // L, unroll=8)
      def _(i):
        d16 = buf[b, pl.ds(i * L, L)]
        m = (d16 >= lo_v) & (d16 < hi_v)
        idx = jnp.where(m, d16 - lo_v, 0)
        plsc.addupdate_scatter(acc, [idx], ones, mask=m)

    start_fetch(0, 0)

    def pair_body(jp, _):
      for b in range(2):
        j = jp * 2 + b

        @pl.when(j + 1 < NCHUNK)
        def _():
          start_fetch(1 - b, j + 1)

        wait_fetch(b, j)
        inner(b)
      return 0

    lax.fori_loop(0, NCHUNK // 2, pair_body, 0)
    pltpu.sync_copy(acc, out_hbm.at[pl.ds(lo, DEG_ROWS)])

  @pl.when(grp == 0)
  def _():
    scan(di_hbm, degi_hbm)

  @pl.when(grp == 1)
  def _():
    scan(du_hbm, degu_hbm)


BLK = 1000


def _tc_layer_body(hu, hi, mi, mu, degi, degu, Wfs, bfs, Wbs, bbs, Wtu, btu,
                   Wti, bti, Wcu, bcu, Wci, bci, out_u, out_i):
  prec = lax.Precision.HIGHEST

  def mm(a, w):
    return jnp.dot(a, w[...], preferred_element_type=jnp.float32,
                   precision=prec)

  hu_b = hu[...]
  hi_b = hi[...]
  inv_i = 1.0 / jnp.maximum(degi[...], 1.0)
  inv_u = 1.0 / jnp.maximum(degu[...], 1.0)

  fi0 = mm(jnp.concatenate([mi[...] * inv_i, hi_b], axis=1), Wfs) + bfs[...]
  bu0 = mm(jnp.concatenate([mu[...] * inv_u, hu_b], axis=1), Wbs) + bbs[...]
  tu = mm(jnp.concatenate([jax.nn.relu(hu_b), jax.nn.relu(bu0)], axis=1),
          Wtu) + btu[...]
  ti = mm(jnp.concatenate([jax.nn.relu(fi0), jax.nn.relu(hi_b)], axis=1),
          Wti) + bti[...]
  out_u[...] = mm(tu, Wcu) + bcu[...]
  out_i[...] = mm(ti, Wci) + bci[...]


def _tc_layer(hu, hi, mi, mu, degi, degu, weights):
  nblk = pl.BlockSpec((BLK, D), lambda j: (j, 0))
  dspec = pl.BlockSpec((BLK, 1), lambda j: (j, 0))
  in_specs = [nblk, nblk, nblk, nblk, dspec, dspec] + [
      pl.BlockSpec(w.shape, lambda j: (0, 0)) for w in weights
  ]
  return pl.pallas_call(
      _tc_layer_body,
      grid=(N // BLK,),
      in_specs=in_specs,
      out_specs=[nblk, nblk],
      out_shape=[
          jax.ShapeDtypeStruct((N, D), jnp.float32),
          jax.ShapeDtypeStruct((N, D), jnp.float32),
      ],
  )(hu, hi, mi, mu, degi, degu, *weights)


def _pack_cols(x):
  xb = x.astype(jnp.bfloat16).reshape(N, D // 2, 2)
  return lax.bitcast_convert_type(xb, jnp.int32).T


def kernel(x_user, x_item, ei_u2i, ei_i2u, Wc_f, bc_f, Ws_f, bs_f, Wc_b, bc_b,
           Ws_b, bs_b, Wt_f, bt_f, Wt_b, bt_b, Wcat, bcat):
  si, di = ei_u2i[0], ei_u2i[1]
  su, du = ei_i2u[0], ei_i2u[1]
  e_u2i = si | lax.shift_left(di, 16)
  e_i2u = su | lax.shift_left(du, 16)

  degi_p, degu_p = _make_degrees()(di, du)
  degi = degi_p[:N].reshape(N, 1)
  degu = degu_p[:N].reshape(N, 1)

  hu, hi = x_user, x_item
  for k in range(K):
    segsum_t = _make_segsum_t()
    mi = segsum_t(_pack_cols(hu), e_u2i).T
    mu = segsum_t(_pack_cols(hi), e_i2u).T
    zz = jnp.zeros((D, D), jnp.float32)
    weights = [
        jnp.concatenate([Wc_f[k], Ws_f[k]], axis=0),
        (bc_f[k] + bs_f[k]).reshape(1, D),
        jnp.concatenate([Wc_b[k], Ws_b[k]], axis=0),
        (bc_b[k] + bs_b[k]).reshape(1, D),
        jnp.block([[Wt_f[k, 0], zz], [zz, Wt_b[k, 0]]]),
        jnp.concatenate([bt_f[k, 0], bt_b[k, 0]]).reshape(1, 2 * D),
        jnp.block([[Wt_f[k, 1], zz], [zz, Wt_b[k, 1]]]),
        jnp.concatenate([bt_f[k, 1], bt_b[k, 1]]).reshape(1, 2 * D),
        Wcat[k, 0], bcat[k, 0].reshape(1, D),
        Wcat[k, 1], bcat[k, 1].reshape(1, D),
    ]
    hu, hi = _tc_layer(hu, hi, mi, mu, degi, degu, weights)
  return jnp.stack([hu, hi])


# speedup vs baseline: 1.5533x; 1.0790x over previous
import functools

import jax
import jax.numpy as jnp
from jax import lax
from jax.experimental import pallas as pl
from jax.experimental.pallas import tpu as pltpu
from jax.experimental.pallas import tpu_sc as plsc

N = 10000
E = 320000
D = 128
K = 2

NC = 2
NS = 16
NW = NC * NS
L = 16

CPW = D // NW
CHUNK = 10000
NCHUNK = E // CHUNK

DEG_ROWS = 640
DEG_PAD = 16 * DEG_ROWS

def _worker_id():
  return lax.axis_index("s") * NC + lax.axis_index("c")


PPW = CPW // 2


@functools.lru_cache(maxsize=None)
def _make_segsum_t():
  mesh = plsc.VectorSubcoreMesh(core_axis_name="c", subcore_axis_name="s")
  return functools.partial(
      pl.kernel,
      out_type=jax.ShapeDtypeStruct((D, N), jnp.float32),
      mesh=mesh,
      scratch_types=[
          pltpu.VMEM((PPW, N), jnp.int32),
          pltpu.VMEM((CPW, N), jnp.float32),
          pltpu.VMEM((2, CHUNK), jnp.int32),
          pltpu.SemaphoreType.DMA,
      ],
      compiler_params=pltpu.CompilerParams(needs_layout_passes=False,
                                           use_tc_tiling_on_sc=False),
  )(_segsum_t_body)


def _segsum_t_body(xp_hbm, edge_hbm, out_hbm, xs, acc, ebuf, esem):
  wid = _worker_id()
  c0 = wid * CPW

  pltpu.sync_copy(xp_hbm.at[pl.ds(wid * PPW, PPW), :], xs)

  zero = jnp.zeros((L,), jnp.float32)

  def zbody(i, _):
    for c in range(CPW):
      acc[c, pl.ds(i * L, L)] = zero
    return 0

  lax.fori_loop(0, N // L, zbody, 0)

  def start_fetch(buf, j):
    pltpu.make_async_copy(edge_hbm.at[pl.ds(j * CHUNK, CHUNK)], ebuf.at[buf],
                          esem).start()

  def wait_fetch(buf, j):
    pltpu.make_async_copy(edge_hbm.at[pl.ds(j * CHUNK, CHUNK)], ebuf.at[buf],
                          esem).wait()

  pvecs = [jnp.full((L,), p, jnp.int32) for p in range(PPW)]
  cvecs = [jnp.full((L,), c, jnp.int32) for c in range(CPW)]
  lo16 = jnp.full((L,), 0xFFFF, jnp.int32)
  hi16 = jnp.full((L,), -65536, jnp.int32)

  def inner(buf):
    @plsc.parallel_loop(0, CHUNK // L, unroll=8)
    def _(i):
      w = ebuf[buf, pl.ds(i * L, L)]
      s16 = w & lo16
      d16 = lax.shift_right_logical(w, 16)
      for p in range(PPW):
        g = plsc.load_gather(xs, [pvecs[p], s16])
        vlo = plsc.bitcast(lax.shift_left(g, 16), jnp.float32)
        vhi = plsc.bitcast(g & hi16, jnp.float32)
        plsc.addupdate_scatter(acc, [cvecs[2 * p], d16], vlo)
        plsc.addupdate_scatter(acc, [cvecs[2 * p + 1], d16], vhi)

  start_fetch(0, 0)

  def pair_body(jp, _):
    for b in range(2):
      j = jp * 2 + b

      @pl.when(j + 1 < NCHUNK)
      def _():
        start_fetch(1 - b, j + 1)

      wait_fetch(b, j)
      inner(b)
    return 0

  lax.fori_loop(0, NCHUNK // 2, pair_body, 0)

  pltpu.sync_copy(acc, out_hbm.at[pl.ds(c0, CPW), :])


@functools.lru_cache(maxsize=None)
def _make_degrees():
  mesh = plsc.VectorSubcoreMesh(core_axis_name="c", subcore_axis_name="s")
  return functools.partial(
      pl.kernel,
      out_type=[
          jax.ShapeDtypeStruct((DEG_PAD,), jnp.float32),
          jax.ShapeDtypeStruct((DEG_PAD,), jnp.float32),
      ],
      mesh=mesh,
      scratch_types=[
          pltpu.VMEM((DEG_ROWS,), jnp.float32),
          pltpu.VMEM((2, CHUNK), jnp.int32),
          pltpu.SemaphoreType.DMA,
      ],
      compiler_params=pltpu.CompilerParams(needs_layout_passes=False,
                                           use_tc_tiling_on_sc=False),
  )(_degrees_body)


def _degrees_body(di_hbm, du_hbm, degi_hbm, degu_hbm, acc, buf, sem):
  wid = _worker_id()
  grp = wid // 16
  slot = wid % 16
  lo = slot * DEG_ROWS

  zero = jnp.zeros((L,), jnp.float32)
  ones = jnp.full((L,), 1.0, jnp.float32)
  lo_v = jnp.full((L,), 1, jnp.int32) * lo
  hi_v = lo_v + DEG_ROWS

  def scan(idx_hbm, out_hbm):
    def zbody(i, _):
      acc[pl.ds(i * L, L)] = zero
      return 0

    lax.fori_loop(0, DEG_ROWS // L, zbody, 0)

    def start_fetch(b, j):
      pltpu.make_async_copy(idx_hbm.at[pl.ds(j * CHUNK, CHUNK)], buf.at[b],
                            sem).start()

    def wait_fetch(b, j):
      pltpu.make_async_copy(idx_hbm.at[pl.ds(j * CHUNK, CHUNK)], buf.at[b],
                            sem).wait()

    def inner(b):
      @plsc.parallel_loop(0, CHUNK // L, unroll=8)
      def _(i):
        d16 = buf[b, pl.ds(i * L, L)]
        m = (d16 >= lo_v) & (d16 < hi_v)
        idx = jnp.where(m, d16 - lo_v, 0)
        plsc.addupdate_scatter(acc, [idx], ones, mask=m)

    start_fetch(0, 0)

    def pair_body(jp, _):
      for b in range(2):
        j = jp * 2 + b

        @pl.when(j + 1 < NCHUNK)
        def _():
          start_fetch(1 - b, j + 1)

        wait_fetch(b, j)
        inner(b)
      return 0

    lax.fori_loop(0, NCHUNK // 2, pair_body, 0)
    pltpu.sync_copy(acc, out_hbm.at[pl.ds(lo, DEG_ROWS)])

  @pl.when(grp == 0)
  def _():
    scan(di_hbm, degi_hbm)

  @pl.when(grp == 1)
  def _():
    scan(du_hbm, degu_hbm)


BLK = 1000


def _tc_half_body(h, m, deg, W1, b1, W2, b2, W3, b3, out):
  prec = lax.Precision.HIGHEST

  def mm(a, w):
    return jnp.dot(a, w[...], preferred_element_type=jnp.float32,
                   precision=prec)

  h_b = h[...]
  inv = 1.0 / jnp.maximum(deg[...], 1.0)
  a0 = mm(jnp.concatenate([m[...] * inv, h_b], axis=1), W1) + b1[...]
  t = mm(jnp.concatenate([jax.nn.relu(a0), jax.nn.relu(h_b)], axis=1),
         W2) + b2[...]
  out[...] = mm(t, W3) + b3[...]


def _tc_half(h, m, deg, weights):
  nblk = pl.BlockSpec((BLK, D), lambda j: (j, 0))
  dspec = pl.BlockSpec((BLK, 1), lambda j: (j, 0))
  in_specs = [nblk, nblk, dspec] + [
      pl.BlockSpec(w.shape, lambda j: (0, 0)) for w in weights
  ]
  return pl.pallas_call(
      _tc_half_body,
      grid=(N // BLK,),
      in_specs=in_specs,
      out_specs=nblk,
      out_shape=jax.ShapeDtypeStruct((N, D), jnp.float32),
  )(h, m, deg, *weights)


def _pack_cols(x):
  xb = x.astype(jnp.bfloat16).reshape(N, D // 2, 2)
  return lax.bitcast_convert_type(xb, jnp.int32).T


def kernel(x_user, x_item, ei_u2i, ei_i2u, Wc_f, bc_f, Ws_f, bs_f, Wc_b, bc_b,
           Ws_b, bs_b, Wt_f, bt_f, Wt_b, bt_b, Wcat, bcat):
  si, di = ei_u2i[0], ei_u2i[1]
  su, du = ei_i2u[0], ei_i2u[1]
  e_u2i = si | lax.shift_left(di, 16)
  e_i2u = su | lax.shift_left(du, 16)

  degi_p, degu_p = _make_degrees()(di, du)
  degi = degi_p[:N].reshape(N, 1)
  degu = degu_p[:N].reshape(N, 1)

  zz = jnp.zeros((D, D), jnp.float32)

  def item_weights(k):
    return [
        jnp.concatenate([Wc_f[k], Ws_f[k]], axis=0),
        (bc_f[k] + bs_f[k]).reshape(1, D),
        jnp.block([[Wt_f[k, 1], zz], [zz, Wt_b[k, 1]]]),
        jnp.concatenate([bt_f[k, 1], bt_b[k, 1]]).reshape(1, 2 * D),
        Wcat[k, 1], bcat[k, 1].reshape(1, D),
    ]

  def user_weights(k):
    return [
        jnp.concatenate([Wc_b[k], Ws_b[k]], axis=0),
        (bc_b[k] + bs_b[k]).reshape(1, D),
        jnp.block([[Wt_b[k, 0], zz], [zz, Wt_f[k, 0]]]),
        jnp.concatenate([bt_b[k, 0], bt_f[k, 0]]).reshape(1, 2 * D),
        jnp.concatenate([Wcat[k, 0][D:], Wcat[k, 0][:D]], axis=0),
        bcat[k, 0].reshape(1, D),
    ]

  segsum_t = _make_segsum_t()
  hu0, hi0 = x_user, x_item
  mi1 = segsum_t(_pack_cols(hu0), e_u2i).T
  mu1 = segsum_t(_pack_cols(hi0), e_i2u).T
  hi1 = _tc_half(hi0, mi1, degi, item_weights(0))
  hu1 = _tc_half(hu0, mu1, degu, user_weights(0))
  mu2 = segsum_t(_pack_cols(hi1), e_i2u).T
  mi2 = segsum_t(_pack_cols(hu1), e_u2i).T
  hu2 = _tc_half(hu1, mu2, degu, user_weights(1))
  hi2 = _tc_half(hi1, mi2, degi, item_weights(1))
  return jnp.stack([hu2, hi2])
